# baseline (speedup 1.0000x reference)
import functools

import jax
import jax.numpy as jnp
from jax import lax
from jax.experimental import pallas as pl
from jax.experimental.pallas import tpu as pltpu
from jax.experimental.pallas import tpu_sc as plsc

N_NODES = 10000
N_EDGES = 320000
D = 128

NC = 2
NS = 16
NW = NC * NS
K = 128
NBUF = 4
AGG_DT = jnp.bfloat16
GW = D
N_PAD = 10240
E_PAD = 327680
CHUNKS = E_PAD // (NW * K)
ROWS_PER_TILE = N_PAD // NS


def _sc_aggregate(x, src3, dst3, zrows):
    mesh = plsc.VectorSubcoreMesh(core_axis_name="c", subcore_axis_name="s")

    @functools.partial(
        pl.kernel,
        out_type=(
            jax.ShapeDtypeStruct((NC, N_PAD, D), AGG_DT),
            jax.ShapeDtypeStruct((NW, N_PAD), jnp.float32),
        ),
        mesh=mesh,
        scratch_types=(
            [pltpu.VMEM((K,), jnp.int32) for _ in range(NBUF)]
            + [pltpu.VMEM((K,), jnp.int32) for _ in range(NBUF)]
            + [pltpu.VMEM((K, GW), AGG_DT) for _ in range(NBUF)]
            + [pltpu.VMEM((N_PAD,), jnp.float32)]
            + [pltpu.VMEM_SHARED((N_PAD, D), AGG_DT)]
            + [pltpu.SemaphoreType.DMA for _ in range(NBUF)]
        ),
        compiler_params=pltpu.CompilerParams(needs_layout_passes=False,
                                             use_tc_tiling_on_sc=False),
    )
    def agg(x_hbm, src_hbm, dst_hbm, z_hbm, sums_hbm, cnts_hbm, *scratch):
        srcb = scratch[0:NBUF]
        dstb = scratch[NBUF:2 * NBUF]
        buf = scratch[2 * NBUF:3 * NBUF]
        hist_v = scratch[3 * NBUF]
        sums_shared = scratch[3 * NBUF + 1]
        sem = scratch[3 * NBUF + 2:3 * NBUF + 2 + NBUF]
        cid = lax.axis_index("c")
        sid = lax.axis_index("s")
        wid = cid * NS + sid

        pltpu.sync_copy(z_hbm, sums_shared.at[pl.ds(sid * ROWS_PER_TILE,
                                                    ROWS_PER_TILE)])

        fz = jnp.zeros((16,), jnp.float32)

        def zero_body(i, carry):
            hist_v[pl.ds(i * 16, 16)] = fz
            return carry

        lax.fori_loop(0, N_PAD // 16, zero_body, 0)

        plsc.subcore_barrier()

        ones16 = jnp.ones((16,), jnp.float32)

        def counts(db):
            for v in range(K // 16):
                dt = db[pl.ds(v * 16, 16)]
                plsc.addupdate_scatter(hist_v, [dt], ones16)

        def load_idx(j, b):
            pltpu.sync_copy(src_hbm.at[wid, j], srcb[b])
            pltpu.sync_copy(dst_hbm.at[wid, j], dstb[b])

        def fire_gather(b):
            pltpu.async_copy(x_hbm.at[srcb[b]], buf[b], sem[b])

        def wait_gather(b):
            pltpu.make_async_copy(x_hbm.at[srcb[b]], buf[b], sem[b]).wait()

        for b in range(NBUF - 1):
            load_idx(b, b)
            fire_gather(b)

        def edge_body(t, carry):
            for b in range(NBUF):
                j = NBUF * t + b
                bn = (b + NBUF - 1) % NBUF
                wait_gather(b)

                @pl.when(j + NBUF - 1 < CHUNKS)
                def _():
                    load_idx(j + NBUF - 1, bn)
                    fire_gather(bn)

                pltpu.sync_copy(buf[b], sums_shared.at[dstb[b]], add=True)
                counts(dstb[b])
            return carry

        lax.fori_loop(0, CHUNKS // NBUF, edge_body, 0)

        plsc.subcore_barrier()

        pltpu.sync_copy(
            sums_shared.at[pl.ds(sid * ROWS_PER_TILE, ROWS_PER_TILE)],
            sums_hbm.at[cid, pl.ds(sid * ROWS_PER_TILE, ROWS_PER_TILE)])
        pltpu.sync_copy(hist_v, cnts_hbm.at[wid])

    return agg(x, src3, dst3, zrows)


def _tc_dense(sums, cnts, x, W_l, b_l, W_r):
    BLK = 400
    grid = (N_NODES // BLK,)

    def body(sums_ref, cnts_ref, x_ref, wl_ref, bl_ref, wr_ref, out_ref):
        s = (sums_ref[0].astype(jnp.float32) +
             sums_ref[1].astype(jnp.float32))
        c = jnp.sum(cnts_ref[...], axis=1)
        m = s * (1.0 / jnp.maximum(c, 1.0))[:, None]
        acc = lax.dot_general(m, wl_ref[...], (((1,), (1,)), ((), ())),
                              preferred_element_type=jnp.float32)
        acc = acc + lax.dot_general(x_ref[...], wr_ref[...],
                                    (((1,), (1,)), ((), ())),
                                    preferred_element_type=jnp.float32)
        out_ref[...] = jnp.maximum(acc + bl_ref[...], 0.0)

    return pl.pallas_call(
        body,
        grid=grid,
        in_specs=[
            pl.BlockSpec((NC, BLK, D), lambda i: (0, i, 0)),
            pl.BlockSpec((BLK, NW), lambda i: (i, 0)),
            pl.BlockSpec((BLK, D), lambda i: (i, 0)),
            pl.BlockSpec((D, D), lambda i: (0, 0)),
            pl.BlockSpec((1, D), lambda i: (0, 0)),
            pl.BlockSpec((D, D), lambda i: (0, 0)),
        ],
        out_specs=pl.BlockSpec((BLK, D), lambda i: (i, 0)),
        out_shape=jax.ShapeDtypeStruct((N_NODES, D), jnp.float32),
    )(sums, cnts, x, W_l, b_l, W_r)


def kernel(x, edge_index, W_l, b_l, W_r):
    ei = edge_index.astype(jnp.int32)
    pad = E_PAD - N_EDGES
    src = jnp.concatenate([ei[0], jnp.zeros((pad,), jnp.int32)])
    dst = jnp.concatenate([ei[1], jnp.full((pad,), N_NODES, jnp.int32)])
    src3 = src.reshape(NW, CHUNKS, K)
    dst3 = dst.reshape(NW, CHUNKS, K)
    zrows = jnp.zeros((ROWS_PER_TILE, D), AGG_DT)

    sums, cnts = _sc_aggregate(x.astype(AGG_DT), src3, dst3, zrows)
    return _tc_dense(sums, cnts.T, x, W_l, b_l.reshape(1, D), W_r)


# speedup vs baseline: 1.0046x; 1.0046x over previous
import functools

import jax
import jax.numpy as jnp
from jax import lax
from jax.experimental import pallas as pl
from jax.experimental.pallas import tpu as pltpu
from jax.experimental.pallas import tpu_sc as plsc

N_NODES = 10000
N_EDGES = 320000
D = 128

NC = 2
NS = 16
NW = NC * NS
K = 128
NBUF = 8
AGG_DT = jnp.bfloat16
GW = D
N_PAD = 10240
E_PAD = 327680
CHUNKS = E_PAD // (NW * K)
ROWS_PER_TILE = N_PAD // NS


def _sc_aggregate(x, src3, dst3, zrows):
    mesh = plsc.VectorSubcoreMesh(core_axis_name="c", subcore_axis_name="s")

    @functools.partial(
        pl.kernel,
        out_type=(
            jax.ShapeDtypeStruct((NC, N_PAD, D), AGG_DT),
            jax.ShapeDtypeStruct((NW, N_PAD), jnp.float32),
        ),
        mesh=mesh,
        scratch_types=(
            [pltpu.VMEM((K,), jnp.int32) for _ in range(NBUF)]
            + [pltpu.VMEM((K,), jnp.int32) for _ in range(NBUF)]
            + [pltpu.VMEM((K, GW), AGG_DT) for _ in range(NBUF)]
            + [pltpu.VMEM((N_PAD,), jnp.float32)]
            + [pltpu.VMEM_SHARED((N_PAD, D), AGG_DT)]
            + [pltpu.SemaphoreType.DMA for _ in range(NBUF)]
        ),
        compiler_params=pltpu.CompilerParams(needs_layout_passes=False,
                                             use_tc_tiling_on_sc=False),
    )
    def agg(x_hbm, src_hbm, dst_hbm, z_hbm, sums_hbm, cnts_hbm, *scratch):
        srcb = scratch[0:NBUF]
        dstb = scratch[NBUF:2 * NBUF]
        buf = scratch[2 * NBUF:3 * NBUF]
        hist_v = scratch[3 * NBUF]
        sums_shared = scratch[3 * NBUF + 1]
        sem = scratch[3 * NBUF + 2:3 * NBUF + 2 + NBUF]
        cid = lax.axis_index("c")
        sid = lax.axis_index("s")
        wid = cid * NS + sid

        pltpu.sync_copy(z_hbm, sums_shared.at[pl.ds(sid * ROWS_PER_TILE,
                                                    ROWS_PER_TILE)])

        fz = jnp.zeros((16,), jnp.float32)

        def zero_body(i, carry):
            hist_v[pl.ds(i * 16, 16)] = fz
            return carry

        lax.fori_loop(0, N_PAD // 16, zero_body, 0)

        plsc.subcore_barrier()

        ones16 = jnp.ones((16,), jnp.float32)

        def counts(db):
            for v in range(K // 16):
                dt = db[pl.ds(v * 16, 16)]
                plsc.addupdate_scatter(hist_v, [dt], ones16)

        def load_idx(j, b):
            pltpu.sync_copy(src_hbm.at[wid, j], srcb[b])
            pltpu.sync_copy(dst_hbm.at[wid, j], dstb[b])

        def fire_gather(b):
            pltpu.async_copy(x_hbm.at[srcb[b]], buf[b], sem[b])

        def wait_gather(b):
            pltpu.make_async_copy(x_hbm.at[srcb[b]], buf[b], sem[b]).wait()

        for b in range(NBUF - 1):
            load_idx(b, b)
            fire_gather(b)

        def edge_body(t, carry):
            for b in range(NBUF):
                j = NBUF * t + b
                bn = (b + NBUF - 1) % NBUF
                wait_gather(b)

                @pl.when(j + NBUF - 1 < CHUNKS)
                def _():
                    load_idx(j + NBUF - 1, bn)
                    fire_gather(bn)

                pltpu.sync_copy(buf[b], sums_shared.at[dstb[b]], add=True)
                counts(dstb[b])
            return carry

        lax.fori_loop(0, CHUNKS // NBUF, edge_body, 0)

        plsc.subcore_barrier()

        pltpu.sync_copy(
            sums_shared.at[pl.ds(sid * ROWS_PER_TILE, ROWS_PER_TILE)],
            sums_hbm.at[cid, pl.ds(sid * ROWS_PER_TILE, ROWS_PER_TILE)])
        pltpu.sync_copy(hist_v, cnts_hbm.at[wid])

    return agg(x, src3, dst3, zrows)


def _tc_dense(sums, cnts, x, W_l, b_l, W_r):
    BLK = 400
    grid = (N_NODES // BLK,)

    def body(sums_ref, cnts_ref, x_ref, wl_ref, bl_ref, wr_ref, out_ref):
        s = (sums_ref[0].astype(jnp.float32) +
             sums_ref[1].astype(jnp.float32))
        c = jnp.sum(cnts_ref[...], axis=1)
        m = s * (1.0 / jnp.maximum(c, 1.0))[:, None]
        acc = lax.dot_general(m, wl_ref[...], (((1,), (1,)), ((), ())),
                              preferred_element_type=jnp.float32)
        acc = acc + lax.dot_general(x_ref[...], wr_ref[...],
                                    (((1,), (1,)), ((), ())),
                                    preferred_element_type=jnp.float32)
        out_ref[...] = jnp.maximum(acc + bl_ref[...], 0.0)

    return pl.pallas_call(
        body,
        grid=grid,
        in_specs=[
            pl.BlockSpec((NC, BLK, D), lambda i: (0, i, 0)),
            pl.BlockSpec((BLK, NW), lambda i: (i, 0)),
            pl.BlockSpec((BLK, D), lambda i: (i, 0)),
            pl.BlockSpec((D, D), lambda i: (0, 0)),
            pl.BlockSpec((1, D), lambda i: (0, 0)),
            pl.BlockSpec((D, D), lambda i: (0, 0)),
        ],
        out_specs=pl.BlockSpec((BLK, D), lambda i: (i, 0)),
        out_shape=jax.ShapeDtypeStruct((N_NODES, D), jnp.float32),
    )(sums, cnts, x, W_l, b_l, W_r)


def kernel(x, edge_index, W_l, b_l, W_r):
    ei = edge_index.astype(jnp.int32)
    pad = E_PAD - N_EDGES
    src = jnp.concatenate([ei[0], jnp.zeros((pad,), jnp.int32)])
    dst = jnp.concatenate([ei[1], jnp.full((pad,), N_NODES, jnp.int32)])
    src3 = src.reshape(NW, CHUNKS, K)
    dst3 = dst.reshape(NW, CHUNKS, K)
    zrows = jnp.zeros((ROWS_PER_TILE, D), AGG_DT)

    sums, cnts = _sc_aggregate(x.astype(AGG_DT), src3, dst3, zrows)
    return _tc_dense(sums, cnts.T, x, W_l, b_l.reshape(1, D), W_r)


# speedup vs baseline: 1.3958x; 1.3894x over previous
import functools

import jax
import jax.numpy as jnp
from jax import lax
from jax.experimental import pallas as pl
from jax.experimental.pallas import tpu as pltpu
from jax.experimental.pallas import tpu_sc as plsc

N_NODES = 10000
N_EDGES = 320000
D = 128

NC = 2
NS = 16
NW = NC * NS
K = 128
NBUF = 4
AGG_DT = jnp.bfloat16
GW = D
N_PAD = 10240
E_PAD = 327680
CHUNKS = E_PAD // (NW * K)
ROWS_PER_TILE = N_PAD // NS


def _sc_aggregate(x, src3, dst3, zrows):
    mesh = plsc.VectorSubcoreMesh(core_axis_name="c", subcore_axis_name="s")

    @functools.partial(
        pl.kernel,
        out_type=(
            jax.ShapeDtypeStruct((NC, N_PAD, D), AGG_DT),
            jax.ShapeDtypeStruct((NW, N_PAD), jnp.float32),
        ),
        mesh=mesh,
        scratch_types=(
            [pltpu.VMEM((K,), jnp.int32) for _ in range(NBUF)]
            + [pltpu.VMEM((K,), jnp.int32) for _ in range(NBUF)]
            + [pltpu.VMEM((K, GW), AGG_DT) for _ in range(NBUF)]
            + [pltpu.VMEM((N_PAD,), jnp.float32)]
            + [pltpu.VMEM_SHARED((N_PAD, D), AGG_DT)]
            + [pltpu.VMEM_SHARED((N_NODES, D), AGG_DT)]
            + [pltpu.SemaphoreType.DMA for _ in range(NBUF)]
        ),
        compiler_params=pltpu.CompilerParams(needs_layout_passes=False,
                                             use_tc_tiling_on_sc=False),
    )
    def agg(x_hbm, src_hbm, dst_hbm, z_hbm, sums_hbm, cnts_hbm, *scratch):
        srcb = scratch[0:NBUF]
        dstb = scratch[NBUF:2 * NBUF]
        buf = scratch[2 * NBUF:3 * NBUF]
        hist_v = scratch[3 * NBUF]
        sums_shared = scratch[3 * NBUF + 1]
        x_shared = scratch[3 * NBUF + 2]
        sem = scratch[3 * NBUF + 3:3 * NBUF + 3 + NBUF]
        cid = lax.axis_index("c")
        sid = lax.axis_index("s")
        wid = cid * NS + sid

        pltpu.sync_copy(z_hbm, sums_shared.at[pl.ds(sid * ROWS_PER_TILE,
                                                    ROWS_PER_TILE)])
        XR = N_NODES // NS
        pltpu.sync_copy(x_hbm.at[pl.ds(sid * XR, XR)],
                        x_shared.at[pl.ds(sid * XR, XR)])

        fz = jnp.zeros((16,), jnp.float32)

        def zero_body(i, carry):
            hist_v[pl.ds(i * 16, 16)] = fz
            return carry

        lax.fori_loop(0, N_PAD // 16, zero_body, 0)

        plsc.subcore_barrier()

        ones16 = jnp.ones((16,), jnp.float32)

        def counts(db):
            for v in range(K // 16):
                dt = db[pl.ds(v * 16, 16)]
                plsc.addupdate_scatter(hist_v, [dt], ones16)

        def load_idx(j, b):
            pltpu.sync_copy(src_hbm.at[wid, j], srcb[b])
            pltpu.sync_copy(dst_hbm.at[wid, j], dstb[b])

        def _x_for(b):
            return x_hbm if b % 2 == 0 else x_shared

        def fire_gather(b):
            pltpu.async_copy(_x_for(b).at[srcb[b]], buf[b], sem[b])

        def wait_gather(b):
            pltpu.make_async_copy(_x_for(b).at[srcb[b]], buf[b], sem[b]).wait()

        for b in range(NBUF - 1):
            load_idx(b, b)
            fire_gather(b)

        def edge_body(t, carry):
            for b in range(NBUF):
                j = NBUF * t + b
                bn = (b + NBUF - 1) % NBUF
                wait_gather(b)

                @pl.when(j + NBUF - 1 < CHUNKS)
                def _():
                    load_idx(j + NBUF - 1, bn)
                    fire_gather(bn)

                pltpu.sync_copy(buf[b], sums_shared.at[dstb[b]], add=True)
                counts(dstb[b])
            return carry

        lax.fori_loop(0, CHUNKS // NBUF, edge_body, 0)

        plsc.subcore_barrier()

        pltpu.sync_copy(
            sums_shared.at[pl.ds(sid * ROWS_PER_TILE, ROWS_PER_TILE)],
            sums_hbm.at[cid, pl.ds(sid * ROWS_PER_TILE, ROWS_PER_TILE)])
        pltpu.sync_copy(hist_v, cnts_hbm.at[wid])

    return agg(x, src3, dst3, zrows)


def _tc_dense(sums, cnts, x, W_l, b_l, W_r):
    BLK = 400
    grid = (N_NODES // BLK,)

    def body(sums_ref, cnts_ref, x_ref, wl_ref, bl_ref, wr_ref, out_ref):
        s = (sums_ref[0].astype(jnp.float32) +
             sums_ref[1].astype(jnp.float32))
        c = jnp.sum(cnts_ref[...], axis=1)
        m = s * (1.0 / jnp.maximum(c, 1.0))[:, None]
        acc = lax.dot_general(m, wl_ref[...], (((1,), (1,)), ((), ())),
                              preferred_element_type=jnp.float32)
        acc = acc + lax.dot_general(x_ref[...], wr_ref[...],
                                    (((1,), (1,)), ((), ())),
                                    preferred_element_type=jnp.float32)
        out_ref[...] = jnp.maximum(acc + bl_ref[...], 0.0)

    return pl.pallas_call(
        body,
        grid=grid,
        in_specs=[
            pl.BlockSpec((NC, BLK, D), lambda i: (0, i, 0)),
            pl.BlockSpec((BLK, NW), lambda i: (i, 0)),
            pl.BlockSpec((BLK, D), lambda i: (i, 0)),
            pl.BlockSpec((D, D), lambda i: (0, 0)),
            pl.BlockSpec((1, D), lambda i: (0, 0)),
            pl.BlockSpec((D, D), lambda i: (0, 0)),
        ],
        out_specs=pl.BlockSpec((BLK, D), lambda i: (i, 0)),
        out_shape=jax.ShapeDtypeStruct((N_NODES, D), jnp.float32),
    )(sums, cnts, x, W_l, b_l, W_r)


def kernel(x, edge_index, W_l, b_l, W_r):
    ei = edge_index.astype(jnp.int32)
    pad = E_PAD - N_EDGES
    src = jnp.concatenate([ei[0], jnp.zeros((pad,), jnp.int32)])
    dst = jnp.concatenate([ei[1], jnp.full((pad,), N_NODES, jnp.int32)])
    src3 = src.reshape(NW, CHUNKS, K)
    dst3 = dst.reshape(NW, CHUNKS, K)
    zrows = jnp.zeros((ROWS_PER_TILE, D), AGG_DT)

    sums, cnts = _sc_aggregate(x.astype(AGG_DT), src3, dst3, zrows)
    return _tc_dense(sums, cnts.T, x, W_l, b_l.reshape(1, D), W_r)


# speedup vs baseline: 1.4032x; 1.0053x over previous
import functools

import jax
import jax.numpy as jnp
from jax import lax
from jax.experimental import pallas as pl
from jax.experimental.pallas import tpu as pltpu
from jax.experimental.pallas import tpu_sc as plsc

N_NODES = 10000
N_EDGES = 320000
D = 128

NC = 2
NS = 16
NW = NC * NS
K = 128
NBUF = 4
AGG_DT = jnp.bfloat16
GW = D
N_PAD = 10240
E_PAD = 327680
CHUNKS = E_PAD // (NW * K)
ROWS_PER_TILE = N_PAD // NS


def _sc_aggregate(x, src3, dst3, zrows):
    mesh = plsc.VectorSubcoreMesh(core_axis_name="c", subcore_axis_name="s")

    @functools.partial(
        pl.kernel,
        out_type=(
            jax.ShapeDtypeStruct((NC, N_PAD, D), AGG_DT),
            jax.ShapeDtypeStruct((NW, N_PAD), jnp.float32),
        ),
        mesh=mesh,
        scratch_types=(
            [pltpu.VMEM((K,), jnp.int32) for _ in range(NBUF)]
            + [pltpu.VMEM((K,), jnp.int32) for _ in range(NBUF)]
            + [pltpu.VMEM((K, GW), AGG_DT) for _ in range(NBUF)]
            + [pltpu.VMEM((N_PAD,), jnp.float32)]
            + [pltpu.VMEM_SHARED((N_PAD, D), AGG_DT)]
            + [pltpu.VMEM_SHARED((N_NODES, D), AGG_DT)]
            + [pltpu.SemaphoreType.DMA for _ in range(NBUF)]
        ),
        compiler_params=pltpu.CompilerParams(needs_layout_passes=False,
                                             use_tc_tiling_on_sc=False),
    )
    def agg(x_hbm, src_hbm, dst_hbm, z_hbm, sums_hbm, cnts_hbm, *scratch):
        srcb = scratch[0:NBUF]
        dstb = scratch[NBUF:2 * NBUF]
        buf = scratch[2 * NBUF:3 * NBUF]
        hist_v = scratch[3 * NBUF]
        sums_shared = scratch[3 * NBUF + 1]
        x_shared = scratch[3 * NBUF + 2]
        sem = scratch[3 * NBUF + 3:3 * NBUF + 3 + NBUF]
        cid = lax.axis_index("c")
        sid = lax.axis_index("s")
        wid = cid * NS + sid

        pltpu.sync_copy(z_hbm, sums_shared.at[pl.ds(sid * ROWS_PER_TILE,
                                                    ROWS_PER_TILE)])
        XR = N_NODES // NS
        pltpu.sync_copy(x_hbm.at[pl.ds(sid * XR, XR)],
                        x_shared.at[pl.ds(sid * XR, XR)])

        fz = jnp.zeros((16,), jnp.float32)

        def zero_body(i, carry):
            hist_v[pl.ds(i * 16, 16)] = fz
            return carry

        lax.fori_loop(0, N_PAD // 16, zero_body, 0)

        plsc.subcore_barrier()

        ones16 = jnp.ones((16,), jnp.float32)

        def counts(db):
            for v in range(K // 16):
                dt = db[pl.ds(v * 16, 16)]
                plsc.addupdate_scatter(hist_v, [dt], ones16)

        def load_idx(j, b):
            pltpu.sync_copy(src_hbm.at[wid, j], srcb[b])
            pltpu.sync_copy(dst_hbm.at[wid, j], dstb[b])

        def _x_for(b):
            return x_hbm if b % 2 == 0 else x_shared

        def fire_gather(b):
            pltpu.async_copy(_x_for(b).at[srcb[b]], buf[b], sem[b])

        def wait_gather(b):
            pltpu.make_async_copy(_x_for(b).at[srcb[b]], buf[b], sem[b]).wait()

        for b in range(NBUF - 1):
            load_idx(b, b)
            fire_gather(b)

        def edge_body(t, carry):
            for b in range(NBUF):
                j = NBUF * t + b
                bn = (b + NBUF - 1) % NBUF
                wait_gather(b)

                @pl.when(j + NBUF - 1 < CHUNKS)
                def _():
                    load_idx(j + NBUF - 1, bn)
                    fire_gather(bn)

                pltpu.sync_copy(buf[b], sums_shared.at[dstb[b]], add=True)
                counts(dstb[b])
            return carry

        lax.fori_loop(0, CHUNKS // NBUF, edge_body, 0)

        plsc.subcore_barrier()

        pltpu.sync_copy(
            sums_shared.at[pl.ds(sid * ROWS_PER_TILE, ROWS_PER_TILE)],
            sums_hbm.at[cid, pl.ds(sid * ROWS_PER_TILE, ROWS_PER_TILE)])
        pltpu.sync_copy(hist_v, cnts_hbm.at[wid])

    return agg(x, src3, dst3, zrows)


def _tc_dense(sums, cnts, x, W_l, b_l, W_r):
    BLK = 512
    grid = (pl.cdiv(N_NODES, BLK),)

    def body(sums_ref, cnts_ref, x_ref, wl_ref, bl_ref, wr_ref, out_ref):
        s = (sums_ref[0].astype(jnp.float32) +
             sums_ref[1].astype(jnp.float32))
        c = jnp.sum(cnts_ref[...], axis=0)
        m = s * (1.0 / jnp.maximum(c, 1.0))[:, None]
        acc = lax.dot_general(m, wl_ref[...], (((1,), (1,)), ((), ())),
                              preferred_element_type=jnp.float32)
        acc = acc + lax.dot_general(x_ref[...], wr_ref[...],
                                    (((1,), (1,)), ((), ())),
                                    preferred_element_type=jnp.float32)
        out_ref[...] = jnp.maximum(acc + bl_ref[...], 0.0)

    return pl.pallas_call(
        body,
        grid=grid,
        in_specs=[
            pl.BlockSpec((NC, BLK, D), lambda i: (0, i, 0)),
            pl.BlockSpec((NW, BLK), lambda i: (0, i)),
            pl.BlockSpec((BLK, D), lambda i: (i, 0)),
            pl.BlockSpec((D, D), lambda i: (0, 0)),
            pl.BlockSpec((1, D), lambda i: (0, 0)),
            pl.BlockSpec((D, D), lambda i: (0, 0)),
        ],
        out_specs=pl.BlockSpec((BLK, D), lambda i: (i, 0)),
        out_shape=jax.ShapeDtypeStruct((N_NODES, D), jnp.float32),
    )(sums, cnts, x, W_l, b_l, W_r)


def kernel(x, edge_index, W_l, b_l, W_r):
    ei = edge_index.astype(jnp.int32)
    pad = E_PAD - N_EDGES
    src = jnp.concatenate([ei[0], jnp.zeros((pad,), jnp.int32)])
    dst = jnp.concatenate([ei[1], jnp.full((pad,), N_NODES, jnp.int32)])
    src3 = src.reshape(NW, CHUNKS, K)
    dst3 = dst.reshape(NW, CHUNKS, K)
    zrows = jnp.zeros((ROWS_PER_TILE, D), AGG_DT)

    sums, cnts = _sc_aggregate(x.astype(AGG_DT), src3, dst3, zrows)
    return _tc_dense(sums, cnts, x, W_l, b_l.reshape(1, D), W_r)
